# Initial kernel scaffold; baseline (speedup 1.0000x reference)
import jax
import jax.numpy as jnp
from jax.experimental import pallas as pl


def kernel(h, edge_index, e, W1, b1, W2, b2, bias, gamma, beta):
    raise NotImplementedError("write your pallas kernel here")


# speedup vs baseline: 1.1208x; 1.1208x over previous
import functools

import jax
import jax.numpy as jnp
from jax import lax
from jax.experimental import pallas as pl
from jax.experimental.pallas import tpu as pltpu
from jax.experimental.pallas import tpu_sc as plsc

N = 10000
E = 160000
IN_DIM = 32
OUT_DIM = 32
E_DIM = 6
EDGE_H = 256
EPS = 1e-5

ROW = 48
BLK = 640
NB = E // BLK

NC = 2
NS = 16
NW = NC * NS
EPW = E // NW
CW = 40
CH = EPW // CW
NTI = 10
NPT = N // NTI


def _sc_gather(h, src3):
    mesh = plsc.VectorSubcoreMesh(core_axis_name="c", subcore_axis_name="s")

    @functools.partial(
        pl.kernel,
        out_type=jax.ShapeDtypeStruct((E, IN_DIM), jnp.float32),
        mesh=mesh,
        scratch_types=[
            pltpu.VMEM((CH, CW), jnp.int32),
            pltpu.VMEM((CW, IN_DIM), jnp.float32),
            pltpu.SemaphoreType.DMA,
        ],
        compiler_params=pltpu.CompilerParams(use_tc_tiling_on_sc=False),
    )
    def k(h_hbm, src_hbm, out_hbm, idx_v, rows_v, sem):
        wid = lax.axis_index("s") * NC + lax.axis_index("c")
        base = wid * EPW
        pltpu.sync_copy(src_hbm.at[wid], idx_v)

        def body(j, carry):
            pltpu.async_copy(h_hbm.at[idx_v.at[j]], rows_v, sem).wait()
            pltpu.sync_copy(rows_v, out_hbm.at[pl.ds(base + j * CW, CW)])
            return carry

        lax.fori_loop(0, CH, body, 0)

    return k(h, src3)


def _sc_scatter(msg48, dst3, zeros):
    mesh = plsc.VectorSubcoreMesh(core_axis_name="c", subcore_axis_name="s")

    @functools.partial(
        pl.kernel,
        out_type=jax.ShapeDtypeStruct((NC * N, ROW), jnp.float32),
        mesh=mesh,
        scratch_types=[
            pltpu.VMEM((CH, CW), jnp.int32),
            pltpu.VMEM((CW, ROW), jnp.float32),
            pltpu.VMEM_SHARED((N, ROW), jnp.float32),
            pltpu.SemaphoreType.DMA,
        ],
        compiler_params=pltpu.CompilerParams(use_tc_tiling_on_sc=False),
    )
    def k(msg_hbm, dst_hbm, zero_hbm, out_hbm, idx_v, vals_v, acc, sem):
        c = lax.axis_index("c")
        s = lax.axis_index("s")
        wid = s * NC + c
        base = wid * EPW
        @pl.when(s < NTI)
        def _():
            pltpu.sync_copy(zero_hbm.at[pl.ds(s * NPT, NPT)],
                            acc.at[pl.ds(s * NPT, NPT)])
        pltpu.sync_copy(dst_hbm.at[wid], idx_v)
        plsc.subcore_barrier()

        def body(j, carry):
            pltpu.sync_copy(msg_hbm.at[pl.ds(base + j * CW, CW)], vals_v)
            pltpu.sync_copy(vals_v, acc.at[idx_v.at[j]], add=True)
            return carry

        lax.fori_loop(0, CH, body, 0)
        plsc.subcore_barrier()

        @pl.when(s < NTI)
        def _():
            pltpu.sync_copy(acc.at[pl.ds(s * NPT, NPT)],
                            out_hbm.at[pl.ds(c * N + s * NPT, NPT)])

    return k(msg48, dst3, zeros)


def _tc_edge_body(e_ref, hs_ref, w1_ref, b1_ref, w2_ref, b2_ref, o_ref):
    hid = jnp.dot(e_ref[...], w1_ref[...], preferred_element_type=jnp.float32)
    hid = jnp.maximum(hid + b1_ref[...], 0.0)
    p = jnp.dot(hid, w2_ref[...], preferred_element_type=jnp.float32)
    p = p + b2_ref[...]
    hs = hs_ref[...]
    acc = p[:, 0:OUT_DIM] * hs[:, 0:1]
    for i in range(1, IN_DIM):
        acc = acc + p[:, i * OUT_DIM:(i + 1) * OUT_DIM] * hs[:, i:i + 1]
    o_ref[:, 0:OUT_DIM] = acc
    o_ref[:, OUT_DIM:OUT_DIM + 1] = jnp.ones((BLK, 1), jnp.float32)
    o_ref[:, OUT_DIM + 1:ROW] = jnp.zeros((BLK, ROW - OUT_DIM - 1), jnp.float32)


def _tc_edge(e, h_src, W1, b1, W2, b2):
    return pl.pallas_call(
        _tc_edge_body,
        grid=(NB,),
        in_specs=[
            pl.BlockSpec((BLK, E_DIM), lambda i: (i, 0)),
            pl.BlockSpec((BLK, IN_DIM), lambda i: (i, 0)),
            pl.BlockSpec((E_DIM, EDGE_H), lambda i: (0, 0)),
            pl.BlockSpec((1, EDGE_H), lambda i: (0, 0)),
            pl.BlockSpec((EDGE_H, IN_DIM * OUT_DIM), lambda i: (0, 0)),
            pl.BlockSpec((1, IN_DIM * OUT_DIM), lambda i: (0, 0)),
        ],
        out_specs=pl.BlockSpec((BLK, ROW), lambda i: (i, 0)),
        out_shape=jax.ShapeDtypeStruct((E, ROW), jnp.float32),
    )(e, h_src, W1, b1.reshape(1, EDGE_H), W2, b2.reshape(1, IN_DIM * OUT_DIM))


def _tc_final_body(parts_ref, bias_ref, gamma_ref, beta_ref, y_ref):
    p0 = parts_ref[0:N, 0:OUT_DIM]
    p1 = parts_ref[N:2 * N, 0:OUT_DIM]
    agg = p0 + p1
    deg = parts_ref[0:N, OUT_DIM:OUT_DIM + 1] \
        + parts_ref[N:2 * N, OUT_DIM:OUT_DIM + 1]
    out = agg / jnp.maximum(deg, 1.0) + bias_ref[...]
    out = jnp.maximum(out, 0.0)
    mu = jnp.mean(out, axis=0, keepdims=True)
    ctr = out - mu
    var = jnp.mean(ctr * ctr, axis=0, keepdims=True)
    y_ref[...] = gamma_ref[...] * ctr * lax.rsqrt(var + EPS) + beta_ref[...]


def _tc_final(parts, bias, gamma, beta):
    return pl.pallas_call(
        _tc_final_body,
        grid=(1,),
        in_specs=[
            pl.BlockSpec((NC * N, ROW), lambda i: (0, 0)),
            pl.BlockSpec((1, OUT_DIM), lambda i: (0, 0)),
            pl.BlockSpec((1, OUT_DIM), lambda i: (0, 0)),
            pl.BlockSpec((1, OUT_DIM), lambda i: (0, 0)),
        ],
        out_specs=pl.BlockSpec((N, OUT_DIM), lambda i: (0, 0)),
        out_shape=jax.ShapeDtypeStruct((N, OUT_DIM), jnp.float32),
    )(parts, bias.reshape(1, OUT_DIM), gamma.reshape(1, OUT_DIM),
      beta.reshape(1, OUT_DIM))


def kernel(h, edge_index, e, W1, b1, W2, b2, bias, gamma, beta):
    src3 = edge_index[0].reshape(NW, CH, CW)
    dst3 = edge_index[1].reshape(NW, CH, CW)
    zeros = jnp.zeros((N, ROW), jnp.float32)

    h_src = _sc_gather(h, src3)
    msg48 = _tc_edge(e, h_src, W1, b1, W2, b2)
    parts = _sc_scatter(msg48, dst3, zeros)
    return _tc_final(parts, bias, gamma, beta)


# speedup vs baseline: 2.4042x; 2.1451x over previous
import functools

import jax
import jax.numpy as jnp
from jax import lax
from jax.experimental import pallas as pl
from jax.experimental.pallas import tpu as pltpu
from jax.experimental.pallas import tpu_sc as plsc

N = 10000
E = 160000
IN_DIM = 32
OUT_DIM = 32
E_DIM = 6
EDGE_H = 256
EPS = 1e-5

ROW = 48
BLK = 640
NB = E // BLK

NC = 2
NS = 16
NW = NC * NS
EPW = E // NW
CW = 40
CH = EPW // CW
NTI = 10
NPT = N // NTI


def _sc_gather(h, src3):
    mesh = plsc.VectorSubcoreMesh(core_axis_name="c", subcore_axis_name="s")

    @functools.partial(
        pl.kernel,
        out_type=jax.ShapeDtypeStruct((E, IN_DIM), jnp.float32),
        mesh=mesh,
        scratch_types=[
            pltpu.VMEM((CH, CW), jnp.int32),
            pltpu.VMEM((CW, IN_DIM), jnp.float32),
            pltpu.SemaphoreType.DMA,
        ],
        compiler_params=pltpu.CompilerParams(use_tc_tiling_on_sc=False),
    )
    def k(h_hbm, src_hbm, out_hbm, idx_v, rows_v, sem):
        wid = lax.axis_index("s") * NC + lax.axis_index("c")
        base = wid * EPW
        pltpu.sync_copy(src_hbm.at[wid], idx_v)

        def body(j, carry):
            pltpu.async_copy(h_hbm.at[idx_v.at[j]], rows_v, sem).wait()
            pltpu.sync_copy(rows_v, out_hbm.at[pl.ds(base + j * CW, CW)])
            return carry

        lax.fori_loop(0, CH, body, 0)

    return k(h, src3)


def _sc_scatter(msg48, dst3, zeros):
    mesh = plsc.VectorSubcoreMesh(core_axis_name="c", subcore_axis_name="s")

    @functools.partial(
        pl.kernel,
        out_type=jax.ShapeDtypeStruct((NC * N, ROW), jnp.float32),
        mesh=mesh,
        scratch_types=[
            pltpu.VMEM((CH, CW), jnp.int32),
            pltpu.VMEM((CW, ROW), jnp.float32),
            pltpu.VMEM_SHARED((N, ROW), jnp.float32),
            pltpu.SemaphoreType.DMA,
        ],
        compiler_params=pltpu.CompilerParams(use_tc_tiling_on_sc=False),
    )
    def k(msg_hbm, dst_hbm, zero_hbm, out_hbm, idx_v, vals_v, acc, sem):
        c = lax.axis_index("c")
        s = lax.axis_index("s")
        wid = s * NC + c
        base = wid * EPW
        @pl.when(s < NTI)
        def _():
            pltpu.sync_copy(zero_hbm.at[pl.ds(s * NPT, NPT)],
                            acc.at[pl.ds(s * NPT, NPT)])
        pltpu.sync_copy(dst_hbm.at[wid], idx_v)
        plsc.subcore_barrier()

        def body(j, carry):
            pltpu.sync_copy(msg_hbm.at[pl.ds(base + j * CW, CW)], vals_v)
            pltpu.sync_copy(vals_v, acc.at[idx_v.at[j]], add=True)
            return carry

        lax.fori_loop(0, CH, body, 0)
        plsc.subcore_barrier()

        @pl.when(s < NTI)
        def _():
            pltpu.sync_copy(acc.at[pl.ds(s * NPT, NPT)],
                            out_hbm.at[pl.ds(c * N + s * NPT, NPT)])

    return k(msg48, dst3, zeros)


def _tc_edge_body(e_ref, hs_ref, w1_ref, b1_ref, w2_ref, b2_ref, s_ref, f_ref,
                  o_ref):
    hid = jnp.dot(e_ref[...], w1_ref[...], preferred_element_type=jnp.float32)
    hid = jnp.maximum(hid + b1_ref[...], 0.0)
    p = jnp.dot(hid, w2_ref[...], preferred_element_type=jnp.float32)
    p = p + b2_ref[...]
    mult = jnp.dot(hs_ref[...], s_ref[...], preferred_element_type=jnp.float32)
    msg = jnp.dot(p * mult, f_ref[...], preferred_element_type=jnp.float32)
    o_ref[:, 0:OUT_DIM] = msg
    o_ref[:, OUT_DIM:OUT_DIM + 1] = jnp.ones((BLK, 1), jnp.float32)
    o_ref[:, OUT_DIM + 1:ROW] = jnp.zeros((BLK, ROW - OUT_DIM - 1), jnp.float32)


def _bcast_fold_consts():
    i = jnp.arange(IN_DIM * OUT_DIM) // OUT_DIM
    o = jnp.arange(IN_DIM * OUT_DIM) % OUT_DIM
    S = (jnp.arange(IN_DIM)[:, None] == i[None, :]).astype(jnp.float32)
    F = (o[:, None] == jnp.arange(OUT_DIM)[None, :]).astype(jnp.float32)
    return S, F


def _tc_edge(e, h_src, W1, b1, W2, b2):
    S, F = _bcast_fold_consts()
    return pl.pallas_call(
        _tc_edge_body,
        grid=(NB,),
        in_specs=[
            pl.BlockSpec((BLK, E_DIM), lambda i: (i, 0)),
            pl.BlockSpec((BLK, IN_DIM), lambda i: (i, 0)),
            pl.BlockSpec((E_DIM, EDGE_H), lambda i: (0, 0)),
            pl.BlockSpec((1, EDGE_H), lambda i: (0, 0)),
            pl.BlockSpec((EDGE_H, IN_DIM * OUT_DIM), lambda i: (0, 0)),
            pl.BlockSpec((1, IN_DIM * OUT_DIM), lambda i: (0, 0)),
            pl.BlockSpec((IN_DIM, IN_DIM * OUT_DIM), lambda i: (0, 0)),
            pl.BlockSpec((IN_DIM * OUT_DIM, OUT_DIM), lambda i: (0, 0)),
        ],
        out_specs=pl.BlockSpec((BLK, ROW), lambda i: (i, 0)),
        out_shape=jax.ShapeDtypeStruct((E, ROW), jnp.float32),
    )(e, h_src, W1, b1.reshape(1, EDGE_H), W2, b2.reshape(1, IN_DIM * OUT_DIM),
      S, F)


def _tc_final_body(parts_ref, bias_ref, gamma_ref, beta_ref, y_ref):
    p0 = parts_ref[0:N, 0:OUT_DIM]
    p1 = parts_ref[N:2 * N, 0:OUT_DIM]
    agg = p0 + p1
    deg = parts_ref[0:N, OUT_DIM:OUT_DIM + 1] \
        + parts_ref[N:2 * N, OUT_DIM:OUT_DIM + 1]
    out = agg / jnp.maximum(deg, 1.0) + bias_ref[...]
    out = jnp.maximum(out, 0.0)
    mu = jnp.mean(out, axis=0, keepdims=True)
    ctr = out - mu
    var = jnp.mean(ctr * ctr, axis=0, keepdims=True)
    y_ref[...] = gamma_ref[...] * ctr * lax.rsqrt(var + EPS) + beta_ref[...]


def _tc_final(parts, bias, gamma, beta):
    return pl.pallas_call(
        _tc_final_body,
        grid=(1,),
        in_specs=[
            pl.BlockSpec((NC * N, ROW), lambda i: (0, 0)),
            pl.BlockSpec((1, OUT_DIM), lambda i: (0, 0)),
            pl.BlockSpec((1, OUT_DIM), lambda i: (0, 0)),
            pl.BlockSpec((1, OUT_DIM), lambda i: (0, 0)),
        ],
        out_specs=pl.BlockSpec((N, OUT_DIM), lambda i: (0, 0)),
        out_shape=jax.ShapeDtypeStruct((N, OUT_DIM), jnp.float32),
    )(parts, bias.reshape(1, OUT_DIM), gamma.reshape(1, OUT_DIM),
      beta.reshape(1, OUT_DIM))


def kernel(h, edge_index, e, W1, b1, W2, b2, bias, gamma, beta):
    src3 = edge_index[0].reshape(NW, CH, CW)
    dst3 = edge_index[1].reshape(NW, CH, CW)
    zeros = jnp.zeros((N, ROW), jnp.float32)

    h_src = _sc_gather(h, src3)
    msg48 = _tc_edge(e, h_src, W1, b1, W2, b2)
    parts = _sc_scatter(msg48, dst3, zeros)
    return _tc_final(parts, bias, gamma, beta)


# speedup vs baseline: 2.4155x; 1.0047x over previous
import functools

import jax
import jax.numpy as jnp
from jax import lax
from jax.experimental import pallas as pl
from jax.experimental.pallas import tpu as pltpu
from jax.experimental.pallas import tpu_sc as plsc

N = 10000
E = 160000
IN_DIM = 32
OUT_DIM = 32
E_DIM = 6
EDGE_H = 256
EPS = 1e-5

ROW = 48
BLK = 640
NB = E // BLK

NC = 2
NS = 16
NW = NC * NS
EPW = E // NW
CW = 40
CH = EPW // CW
NTI = 10
NPT = N // NTI


def _sc_gather(h, src3):
    mesh = plsc.VectorSubcoreMesh(core_axis_name="c", subcore_axis_name="s")

    @functools.partial(
        pl.kernel,
        out_type=jax.ShapeDtypeStruct((E, IN_DIM), jnp.float32),
        mesh=mesh,
        scratch_types=[
            pltpu.VMEM((CH, CW), jnp.int32),
            pltpu.VMEM((CW, IN_DIM), jnp.float32),
            pltpu.SemaphoreType.DMA,
        ],
        compiler_params=pltpu.CompilerParams(use_tc_tiling_on_sc=False),
    )
    def k(h_hbm, src_hbm, out_hbm, idx_v, rows_v, sem):
        wid = lax.axis_index("s") * NC + lax.axis_index("c")
        base = wid * EPW
        pltpu.sync_copy(src_hbm.at[wid], idx_v)

        def body(j, carry):
            pltpu.async_copy(h_hbm.at[idx_v.at[j]], rows_v, sem).wait()
            pltpu.sync_copy(rows_v, out_hbm.at[pl.ds(base + j * CW, CW)])
            return carry

        lax.fori_loop(0, CH, body, 0)

    return k(h, src3)


def _sc_scatter(msg48, dst3, zeros):
    mesh = plsc.VectorSubcoreMesh(core_axis_name="c", subcore_axis_name="s")

    @functools.partial(
        pl.kernel,
        out_type=jax.ShapeDtypeStruct((NC * N, ROW), jnp.float32),
        mesh=mesh,
        scratch_types=[
            pltpu.VMEM((CH, CW), jnp.int32),
            pltpu.VMEM((CW, ROW), jnp.float32),
            pltpu.VMEM_SHARED((N, ROW), jnp.float32),
            pltpu.SemaphoreType.DMA,
        ],
        compiler_params=pltpu.CompilerParams(use_tc_tiling_on_sc=False),
    )
    def k(msg_hbm, dst_hbm, zero_hbm, out_hbm, idx_v, vals_v, acc, sem):
        c = lax.axis_index("c")
        s = lax.axis_index("s")
        wid = s * NC + c
        base = wid * EPW
        @pl.when(s < NTI)
        def _():
            pltpu.sync_copy(zero_hbm.at[pl.ds(s * NPT, NPT)],
                            acc.at[pl.ds(s * NPT, NPT)])
        pltpu.sync_copy(dst_hbm.at[wid], idx_v)
        plsc.subcore_barrier()

        def body(j, carry):
            pltpu.sync_copy(msg_hbm.at[pl.ds(base + j * CW, CW)], vals_v)
            pltpu.sync_copy(vals_v, acc.at[idx_v.at[j]], add=True)
            return carry

        lax.fori_loop(0, CH, body, 0)
        plsc.subcore_barrier()

        @pl.when(s < NTI)
        def _():
            pltpu.sync_copy(acc.at[pl.ds(s * NPT, NPT)],
                            out_hbm.at[pl.ds(c * N + s * NPT, NPT)])

    return k(msg48, dst3, zeros)


def _tc_edge_body(e_ref, hs_ref, w1_ref, b1_ref, w2_ref, b2_ref, s_ref, f_ref,
                  o_ref):
    hid = jnp.dot(e_ref[...], w1_ref[...], preferred_element_type=jnp.float32)
    hid = jnp.maximum(hid + b1_ref[...], 0.0).astype(jnp.bfloat16)
    p = jnp.dot(hid, w2_ref[...], preferred_element_type=jnp.float32)
    p = p + b2_ref[...]
    mult = jnp.dot(hs_ref[...].astype(jnp.bfloat16), s_ref[...],
                   preferred_element_type=jnp.float32)
    msg = jnp.dot((p * mult).astype(jnp.bfloat16), f_ref[...],
                  preferred_element_type=jnp.float32)
    o_ref[:, 0:OUT_DIM] = msg
    o_ref[:, OUT_DIM:OUT_DIM + 1] = jnp.ones((BLK, 1), jnp.float32)
    o_ref[:, OUT_DIM + 1:ROW] = jnp.zeros((BLK, ROW - OUT_DIM - 1), jnp.float32)


def _bcast_fold_consts():
    i = jnp.arange(IN_DIM * OUT_DIM) // OUT_DIM
    o = jnp.arange(IN_DIM * OUT_DIM) % OUT_DIM
    S = (jnp.arange(IN_DIM)[:, None] == i[None, :]).astype(jnp.float32)
    F = (o[:, None] == jnp.arange(OUT_DIM)[None, :]).astype(jnp.float32)
    return S, F


def _tc_edge(e, h_src, W1, b1, W2, b2):
    S, F = _bcast_fold_consts()
    return pl.pallas_call(
        _tc_edge_body,
        grid=(NB,),
        in_specs=[
            pl.BlockSpec((BLK, E_DIM), lambda i: (i, 0)),
            pl.BlockSpec((BLK, IN_DIM), lambda i: (i, 0)),
            pl.BlockSpec((E_DIM, EDGE_H), lambda i: (0, 0)),
            pl.BlockSpec((1, EDGE_H), lambda i: (0, 0)),
            pl.BlockSpec((EDGE_H, IN_DIM * OUT_DIM), lambda i: (0, 0)),
            pl.BlockSpec((1, IN_DIM * OUT_DIM), lambda i: (0, 0)),
            pl.BlockSpec((IN_DIM, IN_DIM * OUT_DIM), lambda i: (0, 0)),
            pl.BlockSpec((IN_DIM * OUT_DIM, OUT_DIM), lambda i: (0, 0)),
        ],
        out_specs=pl.BlockSpec((BLK, ROW), lambda i: (i, 0)),
        out_shape=jax.ShapeDtypeStruct((E, ROW), jnp.float32),
    )(e, h_src, W1, b1.reshape(1, EDGE_H),
      W2.astype(jnp.bfloat16), b2.reshape(1, IN_DIM * OUT_DIM),
      S.astype(jnp.bfloat16), F.astype(jnp.bfloat16))


def _tc_final_body(parts_ref, bias_ref, gamma_ref, beta_ref, y_ref):
    p0 = parts_ref[0:N, 0:OUT_DIM]
    p1 = parts_ref[N:2 * N, 0:OUT_DIM]
    agg = p0 + p1
    deg = parts_ref[0:N, OUT_DIM:OUT_DIM + 1] \
        + parts_ref[N:2 * N, OUT_DIM:OUT_DIM + 1]
    out = agg / jnp.maximum(deg, 1.0) + bias_ref[...]
    out = jnp.maximum(out, 0.0)
    mu = jnp.mean(out, axis=0, keepdims=True)
    ctr = out - mu
    var = jnp.mean(ctr * ctr, axis=0, keepdims=True)
    y_ref[...] = gamma_ref[...] * ctr * lax.rsqrt(var + EPS) + beta_ref[...]


def _tc_final(parts, bias, gamma, beta):
    return pl.pallas_call(
        _tc_final_body,
        grid=(1,),
        in_specs=[
            pl.BlockSpec((NC * N, ROW), lambda i: (0, 0)),
            pl.BlockSpec((1, OUT_DIM), lambda i: (0, 0)),
            pl.BlockSpec((1, OUT_DIM), lambda i: (0, 0)),
            pl.BlockSpec((1, OUT_DIM), lambda i: (0, 0)),
        ],
        out_specs=pl.BlockSpec((N, OUT_DIM), lambda i: (0, 0)),
        out_shape=jax.ShapeDtypeStruct((N, OUT_DIM), jnp.float32),
    )(parts, bias.reshape(1, OUT_DIM), gamma.reshape(1, OUT_DIM),
      beta.reshape(1, OUT_DIM))


def kernel(h, edge_index, e, W1, b1, W2, b2, bias, gamma, beta):
    src3 = edge_index[0].reshape(NW, CH, CW)
    dst3 = edge_index[1].reshape(NW, CH, CW)
    zeros = jnp.zeros((N, ROW), jnp.float32)

    h_src = _sc_gather(h, src3)
    msg48 = _tc_edge(e, h_src, W1, b1, W2, b2)
    parts = _sc_scatter(msg48, dst3, zeros)
    return _tc_final(parts, bias, gamma, beta)


# speedup vs baseline: 2.8505x; 1.1801x over previous
import functools

import jax
import jax.numpy as jnp
from jax import lax
from jax.experimental import pallas as pl
from jax.experimental.pallas import tpu as pltpu
from jax.experimental.pallas import tpu_sc as plsc

N = 10000
E = 160000
IN_DIM = 32
OUT_DIM = 32
E_DIM = 6
EDGE_H = 256
EPS = 1e-5

ROW = 48
BLK = 640
NB = E // BLK

NC = 2
NS = 16
NW = NC * NS
EPW = E // NW
CW = 128
CH = 39
TAIL = EPW - CH * CW
LAG = 3
NBUF = 6
NGRP = 6
NTI = 10
NPT = N // NTI


def _sc_gather(h, src3, srct):
    mesh = plsc.VectorSubcoreMesh(core_axis_name="c", subcore_axis_name="s")

    @functools.partial(
        pl.kernel,
        out_type=jax.ShapeDtypeStruct((E, IN_DIM), jnp.float32),
        mesh=mesh,
        scratch_types=[
            pltpu.VMEM((CH, CW), jnp.int32),
            pltpu.VMEM((1, TAIL), jnp.int32),
            [pltpu.VMEM((CW, IN_DIM), jnp.float32)] * NBUF,
            [pltpu.SemaphoreType.DMA] * NBUF,
            [pltpu.SemaphoreType.DMA] * NBUF,
        ],
        compiler_params=pltpu.CompilerParams(use_tc_tiling_on_sc=False),
    )
    def k(h_hbm, src_hbm, srct_hbm, out_hbm, idx_v, idxt_v, rows, gsem, wsem):
        wid = lax.axis_index("s") * NC + lax.axis_index("c")
        base = wid * EPW

        def start_gather(j, b):
            pltpu.async_copy(h_hbm.at[idx_v.at[j]], rows[b], gsem[b])

        def wait_gather(j, b):
            pltpu.make_async_copy(
                h_hbm.at[idx_v.at[j]], rows[b], gsem[b]).wait()

        def start_write(j, b):
            pltpu.async_copy(
                rows[b], out_hbm.at[pl.ds(base + j * CW, CW)], wsem[b])

        def wait_write(j, b):
            pltpu.make_async_copy(
                rows[b], out_hbm.at[pl.ds(base + j * CW, CW)], wsem[b]).wait()

        pltpu.sync_copy(src_hbm.at[wid], idx_v)
        pltpu.sync_copy(srct_hbm.at[wid], idxt_v)
        for d in range(LAG):
            start_gather(d, d)

        def body(g, carry):
            for d in range(NBUF):
                j = g * NBUF + d
                bn = (d + LAG) % NBUF

                @pl.when(j >= LAG)
                def _():
                    wait_write(j - LAG, bn)

                @pl.when(j + LAG < CH)
                def _():
                    start_gather(j + LAG, bn)

                wait_gather(j, d)
                start_write(j, d)
            return carry

        lax.fori_loop(0, NGRP, body, 0)
        for j in range(NGRP * NBUF, CH):
            wait_write(j - LAG, (j - LAG) % NBUF)
            wait_gather(j, j % NBUF)
            start_write(j, j % NBUF)
        for j in range(CH - LAG, CH):
            wait_write(j, j % NBUF)
        pltpu.async_copy(h_hbm.at[idxt_v.at[0]],
                         rows[LAG].at[pl.ds(0, TAIL)], gsem[LAG]).wait()
        pltpu.sync_copy(rows[LAG].at[pl.ds(0, TAIL)],
                        out_hbm.at[pl.ds(base + CH * CW, TAIL)])

    return k(h, src3, srct)


def _sc_scatter(msg48, dst3, dstt, zeros):
    mesh = plsc.VectorSubcoreMesh(core_axis_name="c", subcore_axis_name="s")

    @functools.partial(
        pl.kernel,
        out_type=jax.ShapeDtypeStruct((NC * N, ROW), jnp.float32),
        mesh=mesh,
        scratch_types=[
            pltpu.VMEM((CH, CW), jnp.int32),
            pltpu.VMEM((1, TAIL), jnp.int32),
            [pltpu.VMEM((CW, ROW), jnp.float32)] * NBUF,
            [pltpu.SemaphoreType.DMA] * NBUF,
            [pltpu.SemaphoreType.DMA] * NBUF,
            pltpu.SemaphoreType.DMA,
            pltpu.VMEM_SHARED((N, ROW), jnp.float32),
        ],
        compiler_params=pltpu.CompilerParams(use_tc_tiling_on_sc=False),
    )
    def k(msg_hbm, dst_hbm, dstt_hbm, zero_hbm, out_hbm, idx_v, idxt_v, vals,
          rsem, ssem, zsem, acc):
        c = lax.axis_index("c")
        s = lax.axis_index("s")
        wid = s * NC + c
        base = wid * EPW

        def start_read(j, b):
            pltpu.async_copy(
                msg_hbm.at[pl.ds(base + j * CW, CW)], vals[b], rsem[b])

        def wait_read(j, b):
            pltpu.make_async_copy(
                msg_hbm.at[pl.ds(base + j * CW, CW)], vals[b], rsem[b]).wait()

        def start_scat(j, b):
            pltpu.async_copy(vals[b], acc.at[idx_v.at[j]], ssem[b], add=True)

        def wait_scat(j, b):
            pltpu.make_async_copy(
                vals[b], acc.at[idx_v.at[j]], ssem[b]).wait()

        @pl.when(s < NTI)
        def _():
            pltpu.async_copy(zero_hbm.at[pl.ds(s * NPT, NPT)],
                             acc.at[pl.ds(s * NPT, NPT)], zsem).wait()

        pltpu.sync_copy(dst_hbm.at[wid], idx_v)
        pltpu.sync_copy(dstt_hbm.at[wid], idxt_v)
        plsc.subcore_barrier()
        for d in range(LAG):
            start_read(d, d)

        def body(g, carry):
            for d in range(NBUF):
                j = g * NBUF + d
                bn = (d + LAG) % NBUF

                @pl.when(j >= LAG)
                def _():
                    wait_scat(j - LAG, bn)

                @pl.when(j + LAG < CH)
                def _():
                    start_read(j + LAG, bn)

                wait_read(j, d)
                start_scat(j, d)
            return carry

        lax.fori_loop(0, NGRP, body, 0)
        for j in range(NGRP * NBUF, CH):
            wait_scat(j - LAG, (j - LAG) % NBUF)
            wait_read(j, j % NBUF)
            start_scat(j, j % NBUF)
        for j in range(CH - LAG, CH):
            wait_scat(j, j % NBUF)
        pltpu.sync_copy(msg_hbm.at[pl.ds(base + CH * CW, TAIL)],
                        vals[LAG].at[pl.ds(0, TAIL)])
        pltpu.sync_copy(vals[LAG].at[pl.ds(0, TAIL)],
                        acc.at[idxt_v.at[0]], add=True)
        plsc.subcore_barrier()

        @pl.when(s < NTI)
        def _():
            pltpu.sync_copy(acc.at[pl.ds(s * NPT, NPT)],
                            out_hbm.at[pl.ds(c * N + s * NPT, NPT)])

    return k(msg48, dst3, dstt, zeros)


def _tc_edge_body(e_ref, hs_ref, w1_ref, b1_ref, w2_ref, b2_ref, s_ref, f_ref,
                  o_ref):
    hid = jnp.dot(e_ref[...], w1_ref[...], preferred_element_type=jnp.float32)
    hid = jnp.maximum(hid + b1_ref[...], 0.0).astype(jnp.bfloat16)
    p = jnp.dot(hid, w2_ref[...], preferred_element_type=jnp.float32)
    p = p + b2_ref[...]
    mult = jnp.dot(hs_ref[...].astype(jnp.bfloat16), s_ref[...],
                   preferred_element_type=jnp.float32)
    msg = jnp.dot((p * mult).astype(jnp.bfloat16), f_ref[...],
                  preferred_element_type=jnp.float32)
    o_ref[:, 0:OUT_DIM] = msg
    o_ref[:, OUT_DIM:OUT_DIM + 1] = jnp.ones((BLK, 1), jnp.float32)
    o_ref[:, OUT_DIM + 1:ROW] = jnp.zeros((BLK, ROW - OUT_DIM - 1), jnp.float32)


def _bcast_fold_consts():
    i = jnp.arange(IN_DIM * OUT_DIM) // OUT_DIM
    o = jnp.arange(IN_DIM * OUT_DIM) % OUT_DIM
    S = (jnp.arange(IN_DIM)[:, None] == i[None, :]).astype(jnp.float32)
    F = (o[:, None] == jnp.arange(OUT_DIM)[None, :]).astype(jnp.float32)
    return S, F


def _tc_edge(e, h_src, W1, b1, W2, b2):
    S, F = _bcast_fold_consts()
    return pl.pallas_call(
        _tc_edge_body,
        grid=(NB,),
        in_specs=[
            pl.BlockSpec((BLK, E_DIM), lambda i: (i, 0)),
            pl.BlockSpec((BLK, IN_DIM), lambda i: (i, 0)),
            pl.BlockSpec((E_DIM, EDGE_H), lambda i: (0, 0)),
            pl.BlockSpec((1, EDGE_H), lambda i: (0, 0)),
            pl.BlockSpec((EDGE_H, IN_DIM * OUT_DIM), lambda i: (0, 0)),
            pl.BlockSpec((1, IN_DIM * OUT_DIM), lambda i: (0, 0)),
            pl.BlockSpec((IN_DIM, IN_DIM * OUT_DIM), lambda i: (0, 0)),
            pl.BlockSpec((IN_DIM * OUT_DIM, OUT_DIM), lambda i: (0, 0)),
        ],
        out_specs=pl.BlockSpec((BLK, ROW), lambda i: (i, 0)),
        out_shape=jax.ShapeDtypeStruct((E, ROW), jnp.float32),
    )(e, h_src, W1, b1.reshape(1, EDGE_H),
      W2.astype(jnp.bfloat16), b2.reshape(1, IN_DIM * OUT_DIM),
      S.astype(jnp.bfloat16), F.astype(jnp.bfloat16))


def _tc_final_body(parts_ref, bias_ref, gamma_ref, beta_ref, y_ref):
    p0 = parts_ref[0:N, 0:OUT_DIM]
    p1 = parts_ref[N:2 * N, 0:OUT_DIM]
    agg = p0 + p1
    deg = parts_ref[0:N, OUT_DIM:OUT_DIM + 1] \
        + parts_ref[N:2 * N, OUT_DIM:OUT_DIM + 1]
    out = agg / jnp.maximum(deg, 1.0) + bias_ref[...]
    out = jnp.maximum(out, 0.0)
    mu = jnp.mean(out, axis=0, keepdims=True)
    ctr = out - mu
    var = jnp.mean(ctr * ctr, axis=0, keepdims=True)
    y_ref[...] = gamma_ref[...] * ctr * lax.rsqrt(var + EPS) + beta_ref[...]


def _tc_final(parts, bias, gamma, beta):
    return pl.pallas_call(
        _tc_final_body,
        grid=(1,),
        in_specs=[
            pl.BlockSpec((NC * N, ROW), lambda i: (0, 0)),
            pl.BlockSpec((1, OUT_DIM), lambda i: (0, 0)),
            pl.BlockSpec((1, OUT_DIM), lambda i: (0, 0)),
            pl.BlockSpec((1, OUT_DIM), lambda i: (0, 0)),
        ],
        out_specs=pl.BlockSpec((N, OUT_DIM), lambda i: (0, 0)),
        out_shape=jax.ShapeDtypeStruct((N, OUT_DIM), jnp.float32),
    )(parts, bias.reshape(1, OUT_DIM), gamma.reshape(1, OUT_DIM),
      beta.reshape(1, OUT_DIM))


def kernel(h, edge_index, e, W1, b1, W2, b2, bias, gamma, beta):
    srcw = edge_index[0].reshape(NW, EPW)
    dstw = edge_index[1].reshape(NW, EPW)
    src3 = srcw[:, :CH * CW].reshape(NW, CH, CW)
    srct = srcw[:, CH * CW:].reshape(NW, 1, TAIL)
    dst3 = dstw[:, :CH * CW].reshape(NW, CH, CW)
    dstt = dstw[:, CH * CW:].reshape(NW, 1, TAIL)
    zeros = jnp.zeros((N, ROW), jnp.float32)

    h_src = _sc_gather(h, src3, srct)
    msg48 = _tc_edge(e, h_src, W1, b1, W2, b2)
    parts = _sc_scatter(msg48, dst3, dstt, zeros)
    return _tc_final(parts, bias, gamma, beta)


# speedup vs baseline: 3.9450x; 1.3840x over previous
import functools

import jax
import jax.numpy as jnp
from jax import lax
from jax.experimental import pallas as pl
from jax.experimental.pallas import tpu as pltpu
from jax.experimental.pallas import tpu_sc as plsc

N = 10000
E = 160000
IN_DIM = 32
OUT_DIM = 32
E_DIM = 6
EDGE_H = 256
EPS = 1e-5

ROW = 48
BLK = 640
NB = E // BLK

NC = 2
NS = 16
NW = NC * NS
EPW = E // NW
CW = 128
CH = 39
TAIL = EPW - CH * CW
LAG = 3
NBUF = 6
NGRP = 6
NTI = 10
NPT = N // NTI


def _sc_gather(h, src3, srct):
    mesh = plsc.VectorSubcoreMesh(core_axis_name="c", subcore_axis_name="s")

    @functools.partial(
        pl.kernel,
        out_type=jax.ShapeDtypeStruct((E, IN_DIM), jnp.float32),
        mesh=mesh,
        scratch_types=[
            pltpu.VMEM((CH, CW), jnp.int32),
            pltpu.VMEM((1, TAIL), jnp.int32),
            [pltpu.VMEM((CW, IN_DIM), jnp.float32)] * NBUF,
            [pltpu.SemaphoreType.DMA] * NBUF,
            [pltpu.SemaphoreType.DMA] * NBUF,
        ],
        compiler_params=pltpu.CompilerParams(use_tc_tiling_on_sc=False),
    )
    def k(h_hbm, src_hbm, srct_hbm, out_hbm, idx_v, idxt_v, rows, gsem, wsem):
        wid = lax.axis_index("s") * NC + lax.axis_index("c")
        base = wid * EPW

        def start_gather(j, b):
            pltpu.async_copy(h_hbm.at[idx_v.at[j]], rows[b], gsem[b])

        def wait_gather(j, b):
            pltpu.make_async_copy(
                h_hbm.at[idx_v.at[j]], rows[b], gsem[b]).wait()

        def start_write(j, b):
            pltpu.async_copy(
                rows[b], out_hbm.at[pl.ds(base + j * CW, CW)], wsem[b])

        def wait_write(j, b):
            pltpu.make_async_copy(
                rows[b], out_hbm.at[pl.ds(base + j * CW, CW)], wsem[b]).wait()

        pltpu.sync_copy(src_hbm.at[wid], idx_v)
        pltpu.sync_copy(srct_hbm.at[wid], idxt_v)
        for d in range(LAG):
            start_gather(d, d)

        def body(g, carry):
            for d in range(NBUF):
                j = g * NBUF + d
                bn = (d + LAG) % NBUF

                @pl.when(j >= LAG)
                def _():
                    wait_write(j - LAG, bn)

                @pl.when(j + LAG < CH)
                def _():
                    start_gather(j + LAG, bn)

                wait_gather(j, d)
                start_write(j, d)
            return carry

        lax.fori_loop(0, NGRP, body, 0)
        for j in range(NGRP * NBUF, CH):
            wait_write(j - LAG, (j - LAG) % NBUF)
            wait_gather(j, j % NBUF)
            start_write(j, j % NBUF)
        for j in range(CH - LAG, CH):
            wait_write(j, j % NBUF)
        pltpu.async_copy(h_hbm.at[idxt_v.at[0]],
                         rows[LAG].at[pl.ds(0, TAIL)], gsem[LAG]).wait()
        pltpu.sync_copy(rows[LAG].at[pl.ds(0, TAIL)],
                        out_hbm.at[pl.ds(base + CH * CW, TAIL)])

    return k(h, src3, srct)


def _sc_scatter(msg48, dst3, dstt, zeros):
    mesh = plsc.VectorSubcoreMesh(core_axis_name="c", subcore_axis_name="s")

    @functools.partial(
        pl.kernel,
        out_type=jax.ShapeDtypeStruct((NC * N, ROW), jnp.float32),
        mesh=mesh,
        scratch_types=[
            pltpu.VMEM((CH, CW), jnp.int32),
            pltpu.VMEM((1, TAIL), jnp.int32),
            [pltpu.VMEM((CW, ROW), jnp.float32)] * NBUF,
            [pltpu.SemaphoreType.DMA] * NBUF,
            [pltpu.SemaphoreType.DMA] * NBUF,
            pltpu.SemaphoreType.DMA,
            pltpu.VMEM_SHARED((N, ROW), jnp.float32),
        ],
        compiler_params=pltpu.CompilerParams(use_tc_tiling_on_sc=False),
    )
    def k(msg_hbm, dst_hbm, dstt_hbm, zero_hbm, out_hbm, idx_v, idxt_v, vals,
          rsem, ssem, zsem, acc):
        c = lax.axis_index("c")
        s = lax.axis_index("s")
        wid = s * NC + c
        base = wid * EPW

        def start_read(j, b):
            pltpu.async_copy(
                msg_hbm.at[pl.ds(base + j * CW, CW)], vals[b], rsem[b])

        def wait_read(j, b):
            pltpu.make_async_copy(
                msg_hbm.at[pl.ds(base + j * CW, CW)], vals[b], rsem[b]).wait()

        def start_scat(j, b):
            pltpu.async_copy(vals[b], acc.at[idx_v.at[j]], ssem[b], add=True)

        def wait_scat(j, b):
            pltpu.make_async_copy(
                vals[b], acc.at[idx_v.at[j]], ssem[b]).wait()

        @pl.when(s < NTI)
        def _():
            pltpu.async_copy(zero_hbm.at[pl.ds(s * NPT, NPT)],
                             acc.at[pl.ds(s * NPT, NPT)], zsem).wait()

        pltpu.sync_copy(dst_hbm.at[wid], idx_v)
        pltpu.sync_copy(dstt_hbm.at[wid], idxt_v)
        plsc.subcore_barrier()
        for d in range(LAG):
            start_read(d, d)

        def body(g, carry):
            for d in range(NBUF):
                j = g * NBUF + d
                bn = (d + LAG) % NBUF

                @pl.when(j >= LAG)
                def _():
                    wait_scat(j - LAG, bn)

                @pl.when(j + LAG < CH)
                def _():
                    start_read(j + LAG, bn)

                wait_read(j, d)
                start_scat(j, d)
            return carry

        lax.fori_loop(0, NGRP, body, 0)
        for j in range(NGRP * NBUF, CH):
            wait_scat(j - LAG, (j - LAG) % NBUF)
            wait_read(j, j % NBUF)
            start_scat(j, j % NBUF)
        for j in range(CH - LAG, CH):
            wait_scat(j, j % NBUF)
        pltpu.sync_copy(msg_hbm.at[pl.ds(base + CH * CW, TAIL)],
                        vals[LAG].at[pl.ds(0, TAIL)])
        pltpu.sync_copy(vals[LAG].at[pl.ds(0, TAIL)],
                        acc.at[idxt_v.at[0]], add=True)
        plsc.subcore_barrier()

        @pl.when(s < NTI)
        def _():
            pltpu.sync_copy(acc.at[pl.ds(s * NPT, NPT)],
                            out_hbm.at[pl.ds(c * N + s * NPT, NPT)])

    return k(msg48, dst3, dstt, zeros)


def _tc_edge_body(eT_ref, hs_ref, w1T_ref, b1T_ref, w2T_ref, b2T_ref, o_ref):
    hidT = jnp.dot(w1T_ref[...], eT_ref[...],
                   preferred_element_type=jnp.float32)
    hidT = jnp.maximum(hidT + b1T_ref[...], 0.0).astype(jnp.bfloat16)
    pT = jnp.dot(w2T_ref[...], hidT, preferred_element_type=jnp.float32)
    pT = pT + b2T_ref[...]
    hsT = hs_ref[...].T
    acc = pT[0:OUT_DIM, :] * hsT[0:1, :]
    for i in range(1, IN_DIM):
        acc = acc + pT[i * OUT_DIM:(i + 1) * OUT_DIM, :] * hsT[i:i + 1, :]
    o_ref[:, 0:OUT_DIM] = acc.T
    o_ref[:, OUT_DIM:OUT_DIM + 1] = jnp.ones((BLK, 1), jnp.float32)
    o_ref[:, OUT_DIM + 1:ROW] = jnp.zeros((BLK, ROW - OUT_DIM - 1), jnp.float32)


def _tc_edge(e, h_src, W1, b1, W2, b2):
    return pl.pallas_call(
        _tc_edge_body,
        grid=(NB,),
        in_specs=[
            pl.BlockSpec((E_DIM, BLK), lambda i: (0, i)),
            pl.BlockSpec((BLK, IN_DIM), lambda i: (i, 0)),
            pl.BlockSpec((EDGE_H, E_DIM), lambda i: (0, 0)),
            pl.BlockSpec((EDGE_H, 1), lambda i: (0, 0)),
            pl.BlockSpec((IN_DIM * OUT_DIM, EDGE_H), lambda i: (0, 0)),
            pl.BlockSpec((IN_DIM * OUT_DIM, 1), lambda i: (0, 0)),
        ],
        out_specs=pl.BlockSpec((BLK, ROW), lambda i: (i, 0)),
        out_shape=jax.ShapeDtypeStruct((E, ROW), jnp.float32),
    )(e.T, h_src, W1.T, b1.reshape(EDGE_H, 1),
      W2.T.astype(jnp.bfloat16), b2.reshape(IN_DIM * OUT_DIM, 1))


def _tc_final_body(parts_ref, bias_ref, gamma_ref, beta_ref, y_ref):
    p0 = parts_ref[0:N, 0:OUT_DIM]
    p1 = parts_ref[N:2 * N, 0:OUT_DIM]
    agg = p0 + p1
    deg = parts_ref[0:N, OUT_DIM:OUT_DIM + 1] \
        + parts_ref[N:2 * N, OUT_DIM:OUT_DIM + 1]
    out = agg / jnp.maximum(deg, 1.0) + bias_ref[...]
    out = jnp.maximum(out, 0.0)
    mu = jnp.mean(out, axis=0, keepdims=True)
    ctr = out - mu
    var = jnp.mean(ctr * ctr, axis=0, keepdims=True)
    y_ref[...] = gamma_ref[...] * ctr * lax.rsqrt(var + EPS) + beta_ref[...]


def _tc_final(parts, bias, gamma, beta):
    return pl.pallas_call(
        _tc_final_body,
        grid=(1,),
        in_specs=[
            pl.BlockSpec((NC * N, ROW), lambda i: (0, 0)),
            pl.BlockSpec((1, OUT_DIM), lambda i: (0, 0)),
            pl.BlockSpec((1, OUT_DIM), lambda i: (0, 0)),
            pl.BlockSpec((1, OUT_DIM), lambda i: (0, 0)),
        ],
        out_specs=pl.BlockSpec((N, OUT_DIM), lambda i: (0, 0)),
        out_shape=jax.ShapeDtypeStruct((N, OUT_DIM), jnp.float32),
    )(parts, bias.reshape(1, OUT_DIM), gamma.reshape(1, OUT_DIM),
      beta.reshape(1, OUT_DIM))


def kernel(h, edge_index, e, W1, b1, W2, b2, bias, gamma, beta):
    srcw = edge_index[0].reshape(NW, EPW)
    dstw = edge_index[1].reshape(NW, EPW)
    src3 = srcw[:, :CH * CW].reshape(NW, CH, CW)
    srct = srcw[:, CH * CW:].reshape(NW, 1, TAIL)
    dst3 = dstw[:, :CH * CW].reshape(NW, CH, CW)
    dstt = dstw[:, CH * CW:].reshape(NW, 1, TAIL)
    zeros = jnp.zeros((N, ROW), jnp.float32)

    h_src = _sc_gather(h, src3, srct)
    msg48 = _tc_edge(e, h_src, W1, b1, W2, b2)
    parts = _sc_scatter(msg48, dst3, dstt, zeros)
    return _tc_final(parts, bias, gamma, beta)
